# Initial kernel scaffold; baseline (speedup 1.0000x reference)
import jax
import jax.numpy as jnp
from jax.experimental import pallas as pl


def kernel(x, pos, params, edge_index, batch, pool_perm1, edge_index2, pool_perm2, edge_index3):
    raise NotImplementedError("write your pallas kernel here")


# speedup vs baseline: 1.1046x; 1.1046x over previous
import functools

import jax
import jax.numpy as jnp
from jax.experimental import pallas as pl


def _pick_block(rows, prefer):
    for b in range(min(prefer, rows), 7, -1):
        if rows % b == 0 and b % 8 == 0:
            return b
    raise ValueError(f"no dividing block for {rows}")


def _edge_mlp(fsrc, dpos, W0f, W0d, b0, W1, b1, W2, b2, block=2048):
    E, F = fsrc.shape
    Kd = dpos.shape[1]
    F1 = W1.shape[1]
    F2 = W2.shape[1]

    def body(f_ref, d_ref, w0f_ref, w0d_ref, b0_ref, w1_ref, b1_ref, w2_ref, b2_ref, o_ref):
        h = jnp.dot(f_ref[...], w0f_ref[...], preferred_element_type=jnp.float32)
        h = h + jnp.dot(d_ref[...], w0d_ref[...], preferred_element_type=jnp.float32)
        h = jnp.maximum(h + b0_ref[...], 0.0)
        h = jnp.dot(h, w1_ref[...], preferred_element_type=jnp.float32) + b1_ref[...]
        h = jnp.maximum(h, 0.0)
        o_ref[...] = jnp.dot(h, w2_ref[...], preferred_element_type=jnp.float32) + b2_ref[...]

    block = _pick_block(E, block)
    grid = (E // block,)
    return pl.pallas_call(
        body,
        grid=grid,
        in_specs=[
            pl.BlockSpec((block, F), lambda i: (i, 0)),
            pl.BlockSpec((block, Kd), lambda i: (i, 0)),
            pl.BlockSpec((F, W0f.shape[1]), lambda i: (0, 0)),
            pl.BlockSpec((Kd, W0d.shape[1]), lambda i: (0, 0)),
            pl.BlockSpec((1, W0f.shape[1]), lambda i: (0, 0)),
            pl.BlockSpec((W1.shape[0], F1), lambda i: (0, 0)),
            pl.BlockSpec((1, F1), lambda i: (0, 0)),
            pl.BlockSpec((F1, F2), lambda i: (0, 0)),
            pl.BlockSpec((1, F2), lambda i: (0, 0)),
        ],
        out_specs=pl.BlockSpec((block, F2), lambda i: (i, 0)),
        out_shape=jax.ShapeDtypeStruct((E, F2), jnp.float32),
    )(fsrc, dpos, W0f, W0d, b0[None], W1, b1[None], W2, b2[None])


def _head(g, p):
    def _bn(h, gain, bias):
        m = h.mean(0)
        v = h.var(0)
        return (h - m) / jnp.sqrt(v + 1e-05) * gain + bias

    out = jax.nn.relu(_bn(g, p['bn1_g'], p['bn1_b']))
    out = out @ p['m_W1'] + p['m_b1']
    out = jax.nn.relu(_bn(out, p['bn2_g'], p['bn2_b']))
    out = out @ p['m_W2'] + p['m_b2']
    out = jax.nn.relu(_bn(out, p['bn3_g'], p['bn3_b']))
    out = out @ p['m_W3'] + p['m_b3']
    return out


def _pad_cols(a, k):
    return jnp.pad(a, ((0, 0), (0, k - a.shape[1])))


def _block_conv(feat, posp, edge_index, p, pre, num_nodes, self_feat=None):
    W0 = p[pre + '_W0']
    F = feat.shape[1]
    W0f = W0[:F]
    W0d = _pad_cols(W0[F:].T, 8).T
    args = (p[pre + '_b0'], p[pre + '_W1'], p[pre + '_b1'], p[pre + '_W2'], p[pre + '_b2'])
    src, dst = edge_index[0], edge_index[1]
    dpos = posp[src] - posp[dst]
    h3e = _edge_mlp(feat[src], dpos, W0f, W0d, *args)
    seg = jax.ops.segment_max(h3e, dst, num_segments=num_nodes)
    if self_feat is not None:
        selfh = _edge_mlp(self_feat, jnp.zeros_like(posp), W0f, W0d, *args)
        return jnp.maximum(seg, selfh)
    return jnp.where(jnp.isfinite(seg), seg, 0.0)


def kernel(x, pos, params, edge_index, batch, pool_perm1, edge_index2, pool_perm2, edge_index3):
    p = params
    N = x.shape[0]
    N2 = pool_perm1.shape[0]
    N3 = pool_perm2.shape[0]

    posp = _pad_cols(pos, 8)
    xp = _pad_cols(x, 8)
    W0 = p['b1_W0']
    W0f1 = _pad_cols(W0[:3].T, 8).T

    out1 = _block_conv(xp, posp, edge_index, {**p, 'b1_W0': jnp.concatenate([W0f1, W0[3:]], 0)},
                       'b1', N, self_feat=xp)
    posp2 = posp[pool_perm1]
    out2 = _block_conv(out1[pool_perm1], posp2, edge_index2, p, 'b2', N2)
    posp3 = posp2[pool_perm2]
    out3 = _block_conv(out2[pool_perm2], posp3, edge_index3, p, 'b3', N3)

    batch3 = batch[pool_perm1][pool_perm2]
    g = jax.ops.segment_max(out3, batch3, num_segments=16)
    g = jnp.where(jnp.isfinite(g), g, 0.0)
    return _head(g, p)


# speedup vs baseline: 1.1227x; 1.0164x over previous
import functools

import jax
import jax.numpy as jnp
from jax.experimental import pallas as pl


def _pick_block(rows, prefer):
    for b in range(min(prefer, rows), 7, -1):
        if rows % b == 0 and b % 8 == 0:
            return b
    raise ValueError(f"no dividing block for {rows}")


def _edge_mlp(fsrc, dpos, W0f, W0d, b0, W1, b1, W2, b2, block=4096):
    E, F = fsrc.shape
    Kd = dpos.shape[1]
    F1 = W1.shape[1]
    F2 = W2.shape[1]

    def body(f_ref, d_ref, w0f_ref, w0d_ref, b0_ref, w1_ref, b1_ref, w2_ref, b2_ref, o_ref):
        h = jnp.dot(f_ref[...], w0f_ref[...], preferred_element_type=jnp.float32)
        h = h + jnp.dot(d_ref[...], w0d_ref[...], preferred_element_type=jnp.float32)
        h = jnp.maximum(h + b0_ref[...], 0.0)
        h = jnp.dot(h, w1_ref[...], preferred_element_type=jnp.float32) + b1_ref[...]
        h = jnp.maximum(h, 0.0)
        o_ref[...] = jnp.dot(h, w2_ref[...], preferred_element_type=jnp.float32) + b2_ref[...]

    block = _pick_block(E, block)
    grid = (E // block,)
    return pl.pallas_call(
        body,
        grid=grid,
        in_specs=[
            pl.BlockSpec((block, F), lambda i: (i, 0)),
            pl.BlockSpec((block, Kd), lambda i: (i, 0)),
            pl.BlockSpec((F, W0f.shape[1]), lambda i: (0, 0)),
            pl.BlockSpec((Kd, W0d.shape[1]), lambda i: (0, 0)),
            pl.BlockSpec((1, W0f.shape[1]), lambda i: (0, 0)),
            pl.BlockSpec((W1.shape[0], F1), lambda i: (0, 0)),
            pl.BlockSpec((1, F1), lambda i: (0, 0)),
            pl.BlockSpec((F1, F2), lambda i: (0, 0)),
            pl.BlockSpec((1, F2), lambda i: (0, 0)),
        ],
        out_specs=pl.BlockSpec((block, F2), lambda i: (i, 0)),
        out_shape=jax.ShapeDtypeStruct((E, F2), jnp.float32),
    )(fsrc, dpos, W0f, W0d, b0[None], W1, b1[None], W2, b2[None])


def _head(g, p):
    def _bn(h, gain, bias):
        m = h.mean(0)
        v = h.var(0)
        return (h - m) / jnp.sqrt(v + 1e-05) * gain + bias

    out = jax.nn.relu(_bn(g, p['bn1_g'], p['bn1_b']))
    out = out @ p['m_W1'] + p['m_b1']
    out = jax.nn.relu(_bn(out, p['bn2_g'], p['bn2_b']))
    out = out @ p['m_W2'] + p['m_b2']
    out = jax.nn.relu(_bn(out, p['bn3_g'], p['bn3_b']))
    out = out @ p['m_W3'] + p['m_b3']
    return out


def _pad_cols(a, k):
    return jnp.pad(a, ((0, 0), (0, k - a.shape[1])))


def _block_conv(feat, posp, edge_index, p, pre, num_nodes, self_feat=None):
    W0 = p[pre + '_W0']
    F = feat.shape[1]
    W0f = W0[:F]
    W0d = _pad_cols(W0[F:].T, 8).T
    args = (p[pre + '_b0'], p[pre + '_W1'], p[pre + '_b1'], p[pre + '_W2'], p[pre + '_b2'])
    src, dst = edge_index[0], edge_index[1]
    dpos = posp[src] - posp[dst]
    h3e = _edge_mlp(feat[src], dpos, W0f, W0d, *args)
    seg = jax.ops.segment_max(h3e, dst, num_segments=num_nodes)
    if self_feat is not None:
        selfh = _edge_mlp(self_feat, jnp.zeros_like(posp), W0f, W0d, *args)
        return jnp.maximum(seg, selfh)
    return jnp.where(jnp.isfinite(seg), seg, 0.0)


def kernel(x, pos, params, edge_index, batch, pool_perm1, edge_index2, pool_perm2, edge_index3):
    p = params
    N = x.shape[0]
    N2 = pool_perm1.shape[0]
    N3 = pool_perm2.shape[0]

    posp = _pad_cols(pos, 8)
    xp = _pad_cols(x, 8)
    W0 = p['b1_W0']
    W0f1 = _pad_cols(W0[:3].T, 8).T

    out1 = _block_conv(xp, posp, edge_index, {**p, 'b1_W0': jnp.concatenate([W0f1, W0[3:]], 0)},
                       'b1', N, self_feat=xp)
    posp2 = posp[pool_perm1]
    out2 = _block_conv(out1[pool_perm1], posp2, edge_index2, p, 'b2', N2)
    posp3 = posp2[pool_perm2]
    out3 = _block_conv(out2[pool_perm2], posp3, edge_index3, p, 'b3', N3)

    batch3 = batch[pool_perm1][pool_perm2]
    g = jax.ops.segment_max(out3, batch3, num_segments=16)
    g = jnp.where(jnp.isfinite(g), g, 0.0)
    return _head(g, p)
